# baseline (speedup 1.0000x reference)
import functools

import jax
import jax.numpy as jnp
from jax import lax
from jax.experimental import pallas as pl
from jax.experimental.pallas import tpu as pltpu
from jax.experimental.pallas import tpu_sc as plsc

_NC = 2
_NS = 16
_L = 16

_WIDTH = 4
_CHUNKS = ((0, 1920), (1920, 1920), (3840, 1920), (5760, 1920),
           (7680, 1920), (9600, 400))
_CMAX = 1920
_CTAIL = 400
_ROWGRP = 16


def _sc_body(x_hbm, wpack_hbm, out_hbm, wbuf, xbuf, zbuf, wbuf2, xbuf2, zbuf2):
    batch = x_hbm.shape[0]
    rows_per_worker = batch // (_NC * _NS)
    n_rowgrps = rows_per_worker // _ROWGRP

    wid = lax.axis_index("s") * _NC + lax.axis_index("c")
    row_base = wid * rows_per_worker

    def process_chunk(g0, csz, wb, xb, zb):
        pltpu.sync_copy(wpack_hbm.at[:, pl.ds(g0, csz)], wb)

        def rowgrp_body(rg, carry):
            r0 = row_base + rg * _ROWGRP
            pltpu.sync_copy(x_hbm.at[pl.ds(r0, _ROWGRP), pl.ds(g0, csz)], xb)

            def lane_body(i, carry):
                s = pl.ds(i * _L, _L)
                w1v = [wb[j, s] for j in range(_WIDTH)]
                b1v = [wb[_WIDTH + j, s] for j in range(_WIDTH)]
                w2v = [wb[2 * _WIDTH + j, s] for j in range(_WIDTH)]
                b2v = wb[3 * _WIDTH, s]
                for r in range(_ROWGRP):
                    xv = xb[r, s]
                    acc = b2v
                    for j in range(_WIDTH):
                        h = jnp.maximum(xv * w1v[j] + b1v[j], 0.0)
                        acc = acc + h * w2v[j]
                    zb[r, s] = acc
                return carry

            lax.fori_loop(0, csz // _L, lane_body, 0)
            pltpu.sync_copy(zb, out_hbm.at[pl.ds(r0, _ROWGRP), pl.ds(g0, csz)])
            return carry

        lax.fori_loop(0, n_rowgrps, rowgrp_body, 0)

    for g0, csz in _CHUNKS:
        if csz == _CMAX:
            process_chunk(g0, csz, wbuf, xbuf, zbuf)
        else:
            process_chunk(g0, csz, wbuf2, xbuf2, zbuf2)


def kernel(features, w1, b1, w2, b2, rows1, cols1, rows2, cols2):
    del rows1, cols1, rows2, cols2
    batch, n_genes = features.shape
    width = w1.shape[0] // n_genes
    wpack = jnp.concatenate(
        [
            w1.reshape(n_genes, width).T,
            b1.reshape(n_genes, width).T,
            w2.reshape(n_genes, width).T,
            b2.reshape(1, n_genes),
        ],
        axis=0,
    )

    mesh = plsc.VectorSubcoreMesh(
        core_axis_name="c", subcore_axis_name="s", num_cores=_NC, num_subcores=_NS
    )
    run = functools.partial(
        pl.kernel,
        out_type=jax.ShapeDtypeStruct((batch, n_genes), features.dtype),
        mesh=mesh,
        scratch_types=[
            pltpu.VMEM((3 * width + 1, _CMAX), jnp.float32),
            pltpu.VMEM((_ROWGRP, _CMAX), jnp.float32),
            pltpu.VMEM((_ROWGRP, _CMAX), jnp.float32),
            pltpu.VMEM((3 * width + 1, _CTAIL), jnp.float32),
            pltpu.VMEM((_ROWGRP, _CTAIL), jnp.float32),
            pltpu.VMEM((_ROWGRP, _CTAIL), jnp.float32),
        ],
    )(_sc_body)
    return run(features, wpack)


# speedup vs baseline: 1.2302x; 1.2302x over previous
import functools

import jax
import jax.numpy as jnp
from jax import lax
from jax.experimental import pallas as pl
from jax.experimental.pallas import tpu as pltpu
from jax.experimental.pallas import tpu_sc as plsc

_NC = 2
_NS = 16
_L = 16

_WIDTH = 4
_NW = 13
_CMAX = 1920
_CTAIL = 400
_NCHUNK = 5
_ROWGRP = 8


def _compute_rows(wb, xb, zb, n_lanesteps):
    def lane_body(i, carry):
        s = pl.ds(i * _L, _L)
        w1v = [wb[j, s] for j in range(_WIDTH)]
        b1v = [wb[_WIDTH + j, s] for j in range(_WIDTH)]
        w2v = [wb[2 * _WIDTH + j, s] for j in range(_WIDTH)]
        b2v = wb[3 * _WIDTH, s]
        for r in range(_ROWGRP):
            xv = xb[r, s]
            acc = b2v
            for j in range(_WIDTH):
                h = jnp.maximum(xv * w1v[j] + b1v[j], 0.0)
                acc = acc + h * w2v[j]
            zb[r, s] = acc
        return carry

    lax.fori_loop(0, n_lanesteps, lane_body, 0)


def _sc_body(x_hbm, wpack_hbm, out_hbm,
             wb, xb0, xb1, zb0, zb1, wbt, xbt, zbt,
             wsem, xsem0, xsem1, zsem0, zsem1):
    batch = x_hbm.shape[0]
    rows_per_worker = batch // (_NC * _NS)

    wid = lax.axis_index("s") * _NC + lax.axis_index("c")
    row_base = wid * rows_per_worker

    xbufs, xsems = (xb0, xb1), (xsem0, xsem1)
    zbufs, zsems = (zb0, zb1), (zsem0, zsem1)

    n_rowgrps = rows_per_worker // _ROWGRP
    items = [(gc, rg) for gc in range(_NCHUNK) for rg in range(n_rowgrps)]
    n_items = len(items)

    def start_w(gc):
        return pltpu.async_copy(
            wpack_hbm.at[:, pl.ds(gc * _CMAX, _CMAX)], wb, wsem)

    def start_x(k):
        gc, rg = items[k]
        r0 = row_base + rg * _ROWGRP
        return pltpu.async_copy(
            x_hbm.at[pl.ds(r0, _ROWGRP), pl.ds(gc * _CMAX, _CMAX)],
            xbufs[k % 2], xsems[k % 2])

    def start_z(k):
        gc, rg = items[k]
        r0 = row_base + rg * _ROWGRP
        return pltpu.async_copy(
            zbufs[k % 2],
            out_hbm.at[pl.ds(r0, _ROWGRP), pl.ds(gc * _CMAX, _CMAX)],
            zsems[k % 2])

    w_handles = {0: start_w(0)}
    x_handles = {0: start_x(0)}
    z_handles = {}

    for k in range(n_items):
        gc, rg = items[k]
        if k + 1 < n_items:
            x_handles[k + 1] = start_x(k + 1)
        if rg == 0:
            w_handles[gc].wait()
        x_handles[k].wait()
        if k >= 2:
            z_handles[k - 2].wait()
        _compute_rows(wb, xbufs[k % 2], zbufs[k % 2], _CMAX // _L)
        if k + 1 < n_items and items[k + 1][1] == 0:
            w_handles[items[k + 1][0]] = start_w(items[k + 1][0])
        z_handles[k] = start_z(k)
    z_handles[n_items - 2].wait()
    z_handles[n_items - 1].wait()

    g0 = _NCHUNK * _CMAX
    pltpu.sync_copy(wpack_hbm.at[:, pl.ds(g0, _CTAIL)], wbt)

    def tail_body(rg, carry):
        r0 = row_base + rg * _ROWGRP
        pltpu.sync_copy(x_hbm.at[pl.ds(r0, _ROWGRP), pl.ds(g0, _CTAIL)], xbt)
        _compute_rows(wbt, xbt, zbt, _CTAIL // _L)
        pltpu.sync_copy(zbt, out_hbm.at[pl.ds(r0, _ROWGRP), pl.ds(g0, _CTAIL)])
        return carry

    lax.fori_loop(0, n_rowgrps, tail_body, 0)


def kernel(features, w1, b1, w2, b2, rows1, cols1, rows2, cols2):
    del rows1, cols1, rows2, cols2
    batch, n_genes = features.shape
    width = w1.shape[0] // n_genes
    wpack = jnp.concatenate(
        [
            w1.reshape(n_genes, width).T,
            b1.reshape(n_genes, width).T,
            w2.reshape(n_genes, width).T,
            b2.reshape(1, n_genes),
        ],
        axis=0,
    )

    mesh = plsc.VectorSubcoreMesh(
        core_axis_name="c", subcore_axis_name="s", num_cores=_NC, num_subcores=_NS
    )
    run = functools.partial(
        pl.kernel,
        out_type=jax.ShapeDtypeStruct((batch, n_genes), features.dtype),
        mesh=mesh,
        scratch_types=[
            pltpu.VMEM((_NW, _CMAX), jnp.float32),
            pltpu.VMEM((_ROWGRP, _CMAX), jnp.float32),
            pltpu.VMEM((_ROWGRP, _CMAX), jnp.float32),
            pltpu.VMEM((_ROWGRP, _CMAX), jnp.float32),
            pltpu.VMEM((_ROWGRP, _CMAX), jnp.float32),
            pltpu.VMEM((_NW, _CTAIL), jnp.float32),
            pltpu.VMEM((_ROWGRP, _CTAIL), jnp.float32),
            pltpu.VMEM((_ROWGRP, _CTAIL), jnp.float32),
            pltpu.SemaphoreType.DMA,
            pltpu.SemaphoreType.DMA,
            pltpu.SemaphoreType.DMA,
            pltpu.SemaphoreType.DMA,
            pltpu.SemaphoreType.DMA,
        ],
    )(_sc_body)
    return run(features, wpack)


# speedup vs baseline: 2.4075x; 1.9569x over previous
import jax
import jax.numpy as jnp
from jax.experimental import pallas as pl


def _body(x_ref, w1_ref, b1_ref, w2_ref, b2_ref, o_ref):
    x = x_ref[...]
    width = w1_ref.shape[0]
    acc = jnp.broadcast_to(b2_ref[...], x.shape)
    for j in range(width):
        h = jnp.maximum(x * w1_ref[j : j + 1, :] + b1_ref[j : j + 1, :], 0.0)
        acc = acc + h * w2_ref[j : j + 1, :]
    o_ref[...] = acc


def kernel(features, w1, b1, w2, b2, rows1, cols1, rows2, cols2):
    del rows1, cols1, rows2, cols2
    batch, n_genes = features.shape
    width = w1.shape[0] // n_genes
    w1t = w1.reshape(n_genes, width).T
    b1t = b1.reshape(n_genes, width).T
    w2t = w2.reshape(n_genes, width).T
    b2r = b2.reshape(1, n_genes)

    bt = 128
    grid = (batch // bt,)
    return pl.pallas_call(
        _body,
        grid=grid,
        in_specs=[
            pl.BlockSpec((bt, n_genes), lambda i: (i, 0)),
            pl.BlockSpec((width, n_genes), lambda i: (0, 0)),
            pl.BlockSpec((width, n_genes), lambda i: (0, 0)),
            pl.BlockSpec((width, n_genes), lambda i: (0, 0)),
            pl.BlockSpec((1, n_genes), lambda i: (0, 0)),
        ],
        out_specs=pl.BlockSpec((bt, n_genes), lambda i: (i, 0)),
        out_shape=jax.ShapeDtypeStruct((batch, n_genes), features.dtype),
    )(features, w1t, b1t, w2t, b2r)
